# baseline (speedup 1.0000x reference)
import jax
import jax.numpy as jnp
import numpy as np
from jax import lax
from jax.experimental import pallas as pl
from jax.experimental.pallas import tpu as pltpu

_K = 8192
_D = 32
_IN = 256
_CC = 0.25
_EPS = 1e-5
_TB = 128


def _body(x_ref, emb_ref, w1_ref, b1_ref, w2_ref, b2_ref,
          bn1w_ref, bn1b_ref, bn2w_ref, bn2b_ref,
          qst_ref, counts_ref, losssum_ref):
    i = pl.program_id(0)
    xb = x_ref[...]
    sq = jnp.sqrt(jnp.float32(1.0 + _EPS))
    flat = (xb / sq) * bn1w_ref[...] + bn1b_ref[...]
    h = lax.dot_general(flat, w1_ref[...], (((1,), (1,)), ((), ())),
                        preferred_element_type=jnp.float32) + b1_ref[...]
    emb = emb_ref[...]
    a = jnp.sum(h * h, axis=1, keepdims=True)
    b = jnp.sum(emb * emb, axis=1)
    c = lax.dot_general(h, emb, (((1,), (1,)), ((), ())),
                        preferred_element_type=jnp.float32)
    dist = (a + b[None, :]) - 2.0 * c
    minv = jnp.min(dist, axis=1, keepdims=True)
    iota = lax.broadcasted_iota(jnp.int32, (_TB, _K), 1)
    idx = jnp.min(jnp.where(dist == minv, iota, _K), axis=1)
    onehot = (iota == idx[:, None]).astype(jnp.float32)
    cpart = jnp.sum(onehot, axis=0)[None, :]
    q = lax.dot_general(onehot, emb, (((1,), (0,)), ((), ())),
                        preferred_element_type=jnp.float32)
    qbn = (q / sq) * bn2w_ref[...] + bn2b_ref[...]
    out = lax.dot_general(qbn, w2_ref[...], (((1,), (1,)), ((), ())),
                          preferred_element_type=jnp.float32) + b2_ref[...]
    qst_ref[...] = xb + (out - xb)
    lpart = jnp.sum((out - xb) ** 2)

    @pl.when(i == 0)
    def _init():
        counts_ref[...] = cpart
        losssum_ref[0, 0] = lpart

    @pl.when(i != 0)
    def _acc():
        counts_ref[...] += cpart
        losssum_ref[0, 0] += lpart


def kernel(x, emb, W1, b1, W2, b2, bn1_w, bn1_b, bn2_w, bn2_b):
    shape = x.shape
    T = shape[0] * shape[1]
    xf = x.reshape(T, _IN)
    grid = T // _TB
    full = lambda i: (0, 0)
    qst, counts, losssum = pl.pallas_call(
        _body,
        grid=(grid,),
        in_specs=[
            pl.BlockSpec((_TB, _IN), lambda i: (i, 0)),
            pl.BlockSpec((_K, _D), full),
            pl.BlockSpec((_D, _IN), full),
            pl.BlockSpec((1, _D), full),
            pl.BlockSpec((_IN, _D), full),
            pl.BlockSpec((1, _IN), full),
            pl.BlockSpec((1, _IN), full),
            pl.BlockSpec((1, _IN), full),
            pl.BlockSpec((1, _D), full),
            pl.BlockSpec((1, _D), full),
        ],
        out_specs=[
            pl.BlockSpec((_TB, _IN), lambda i: (i, 0)),
            pl.BlockSpec((1, _K), full),
            pl.BlockSpec(memory_space=pltpu.SMEM),
        ],
        out_shape=[
            jax.ShapeDtypeStruct((T, _IN), jnp.float32),
            jax.ShapeDtypeStruct((1, _K), jnp.float32),
            jax.ShapeDtypeStruct((1, 1), jnp.float32),
        ],
    )(xf, emb, W1, b1[None, :], W2, b2[None, :],
      bn1_w[None, :], bn1_b[None, :], bn2_w[None, :], bn2_b[None, :])
    m = losssum[0, 0] / jnp.float32(T * _IN)
    loss = m + _CC * m
    usage = counts[0] / jnp.float32(T)
    return (loss, qst.reshape(shape), usage, emb)


# speedup vs baseline: 1.2071x; 1.2071x over previous
import jax
import jax.numpy as jnp
import numpy as np
from jax import lax
from jax.experimental import pallas as pl
from jax.experimental.pallas import tpu as pltpu

_K = 8192
_D = 32
_IN = 256
_CC = 0.25
_EPS = 1e-5
_TB = 128


def _body(x_ref, emb_ref, w1_ref, b1_ref, w2_ref, b2_ref,
          bn1w_ref, bn1b_ref, bn2w_ref, bn2b_ref,
          qst_ref, counts_ref, losssum_ref, bsq_ref):
    i = pl.program_id(0)

    @pl.when(i == 0)
    def _precompute():
        e = emb_ref[...]
        bsq_ref[...] = jnp.sum(e * e, axis=1)[None, :]

    xb = x_ref[...]
    sq = jnp.sqrt(jnp.float32(1.0 + _EPS))
    flat = (xb / sq) * bn1w_ref[...] + bn1b_ref[...]
    h = lax.dot_general(flat, w1_ref[...], (((1,), (1,)), ((), ())),
                        preferred_element_type=jnp.float32) + b1_ref[...]
    emb = emb_ref[...]
    a = jnp.sum(h * h, axis=1, keepdims=True)
    c = lax.dot_general(h, emb, (((1,), (1,)), ((), ())),
                        preferred_element_type=jnp.float32)
    dist = (a + bsq_ref[...]) - 2.0 * c
    minv = jnp.min(dist, axis=1, keepdims=True)
    iota = lax.broadcasted_iota(jnp.int32, (_TB, _K), 1)
    idx = jnp.min(jnp.where(dist == minv, iota, _K), axis=1)
    onehot = (iota == idx[:, None]).astype(jnp.float32)
    cpart = jnp.sum(onehot, axis=0)[None, :]
    q = lax.dot_general(onehot, emb, (((1,), (0,)), ((), ())),
                        preferred_element_type=jnp.float32)
    qbn = (q / sq) * bn2w_ref[...] + bn2b_ref[...]
    out = lax.dot_general(qbn, w2_ref[...], (((1,), (1,)), ((), ())),
                          preferred_element_type=jnp.float32) + b2_ref[...]
    qst_ref[...] = xb + (out - xb)
    lpart = jnp.sum((out - xb) ** 2)

    @pl.when(i == 0)
    def _init():
        counts_ref[...] = cpart
        losssum_ref[0, 0] = lpart

    @pl.when(i != 0)
    def _acc():
        counts_ref[...] += cpart
        losssum_ref[0, 0] += lpart


def kernel(x, emb, W1, b1, W2, b2, bn1_w, bn1_b, bn2_w, bn2_b):
    shape = x.shape
    T = shape[0] * shape[1]
    xf = x.reshape(T, _IN)
    grid = T // _TB
    full = lambda i: (0, 0)
    qst, counts, losssum = pl.pallas_call(
        _body,
        grid=(grid,),
        in_specs=[
            pl.BlockSpec((_TB, _IN), lambda i: (i, 0)),
            pl.BlockSpec((_K, _D), full),
            pl.BlockSpec((_D, _IN), full),
            pl.BlockSpec((1, _D), full),
            pl.BlockSpec((_IN, _D), full),
            pl.BlockSpec((1, _IN), full),
            pl.BlockSpec((1, _IN), full),
            pl.BlockSpec((1, _IN), full),
            pl.BlockSpec((1, _D), full),
            pl.BlockSpec((1, _D), full),
        ],
        out_specs=[
            pl.BlockSpec((_TB, _IN), lambda i: (i, 0)),
            pl.BlockSpec((1, _K), full),
            pl.BlockSpec(memory_space=pltpu.SMEM),
        ],
        out_shape=[
            jax.ShapeDtypeStruct((T, _IN), jnp.float32),
            jax.ShapeDtypeStruct((1, _K), jnp.float32),
            jax.ShapeDtypeStruct((1, 1), jnp.float32),
        ],
        scratch_shapes=[pltpu.VMEM((1, _K), jnp.float32)],
    )(xf, emb, W1, b1[None, :], W2, b2[None, :],
      bn1_w[None, :], bn1_b[None, :], bn2_w[None, :], bn2_b[None, :])
    m = losssum[0, 0] / jnp.float32(T * _IN)
    loss = m + _CC * m
    usage = counts[0] / jnp.float32(T)
    return (loss, qst.reshape(shape), usage, emb)
